# baseline (speedup 1.0000x reference)
import functools

import jax
import jax.numpy as jnp
from jax import lax
from jax.experimental import pallas as pl
from jax.experimental.pallas import tpu as pltpu
from jax.experimental.pallas import tpu_sc as plsc

VOCAB = 1000
VOCAB_PAD = 1024
EMBED_DIM = 128
CHUNK = 64


def _table_kernel(e_ref, w_ref, bias_ref, out_ref):
    acc = lax.dot_general(
        e_ref[...],
        w_ref[...],
        dimension_numbers=(((1,), (1,)), ((), ())),
        preferred_element_type=jnp.float32,
        precision=lax.Precision.HIGHEST,
    )
    out_ref[...] = acc + bias_ref[...]


def _make_table(embed_table, W, b):
    w_pad = jnp.zeros((VOCAB_PAD, EMBED_DIM), jnp.float32).at[:VOCAB].set(W)
    b_pad = jnp.zeros((1, VOCAB_PAD), jnp.float32).at[0, :VOCAB].set(b)
    return pl.pallas_call(
        _table_kernel,
        out_shape=jax.ShapeDtypeStruct((VOCAB, VOCAB_PAD), jnp.float32),
    )(embed_table, w_pad, b_pad)


def _gather_fn(n_tokens):
    info = plsc.get_sparse_core_info()
    nc, ns = info.num_cores, info.num_subcores
    nw = nc * ns
    assert n_tokens % (nw * CHUNK) == 0
    nchunk = n_tokens // (nw * CHUNK)
    mesh = plsc.VectorSubcoreMesh(core_axis_name="c", subcore_axis_name="s")

    @functools.partial(
        pl.kernel,
        mesh=mesh,
        out_type=jax.ShapeDtypeStruct((n_tokens, VOCAB_PAD), jnp.float32),
        scratch_types=[
            pltpu.VMEM((nchunk, CHUNK), jnp.int32),
            pltpu.VMEM((CHUNK, VOCAB_PAD), jnp.float32),
            pltpu.SemaphoreType.DMA,
        ],
    )
    def gather(idx_hbm, table_hbm, out_hbm, idx_v, rows_v, sem):
        wid = lax.axis_index("s") * nc + lax.axis_index("c")
        base = wid * (nchunk * CHUNK)
        pltpu.sync_copy(idx_hbm.at[wid], idx_v)

        def body(c, carry):
            pltpu.async_copy(table_hbm.at[idx_v.at[c]], rows_v, sem).wait()
            pltpu.sync_copy(rows_v, out_hbm.at[pl.ds(base + c * CHUNK, CHUNK)])
            return carry

        lax.fori_loop(0, nchunk, body, 0)

    return gather


def kernel(x, embed_table, W, b):
    B, L = x.shape
    n_tokens = B * L
    table = _make_table(embed_table, W, b)
    info = plsc.get_sparse_core_info()
    nw = info.num_cores * info.num_subcores
    idx = x.reshape(nw, n_tokens // (nw * CHUNK), CHUNK).astype(jnp.int32)
    out = _gather_fn(n_tokens)(idx, table)
    return out[:, :VOCAB].reshape(B, L, VOCAB)
